# Initial kernel scaffold; baseline (speedup 1.0000x reference)
import jax
import jax.numpy as jnp
from jax.experimental import pallas as pl


def kernel(inputs, W3, b, velocity, I, J):
    raise NotImplementedError("write your pallas kernel here")


# speedup vs baseline: 1.0716x; 1.0716x over previous
import functools

import jax
import jax.numpy as jnp
from jax import lax
from jax.experimental import pallas as pl
from jax.experimental.pallas import tpu as pltpu, tpu_sc as plsc

B = 4096
N = 16384
NNZ = 12300
LANES = 16
NNZP = ((NNZ + LANES - 1) // LANES) * LANES
CHUNKS = NNZP // LANES

NUM_CORES = 2
NUM_SUBCORES = 16
NW = NUM_CORES * NUM_SUBCORES
ROWS_PER_W = B // NW
PAIRS_PER_W = ROWS_PER_W // 2


def _sc_kernel(x_hbm, w3_hbm, b_hbm, vel_hbm, i_hbm, j_hbm, out_hbm,
               iref, jref, vref, bias_v, x0, x1, o0, o1):
    wid = lax.axis_index("s") * NUM_CORES + lax.axis_index("c")
    base_row = wid * ROWS_PER_W

    pltpu.sync_copy(i_hbm, iref)
    pltpu.sync_copy(j_hbm, jref)
    pltpu.sync_copy(w3_hbm, vref)
    pltpu.sync_copy(vel_hbm, x0)
    pltpu.sync_copy(b_hbm, bias_v)

    def vals_body(c, carry):
        s = pl.ds(c * LANES, LANES)
        g = plsc.load_gather(x0, [jref[s]])
        vref[s] = vref[s] * g
        return carry

    lax.fori_loop(0, CHUNKS, vals_body, 0)

    def row_body(it, carry):
        r0 = base_row + it * 2
        pltpu.sync_copy(x_hbm.at[r0], x0)
        pltpu.sync_copy(x_hbm.at[r0 + 1], x1)

        def bias_body(c, inner):
            s = pl.ds(c * LANES, LANES)
            bv = bias_v[s]
            o0[s] = bv
            o1[s] = bv
            return inner

        lax.fori_loop(0, N // LANES, bias_body, 0, unroll=4)

        def chunk_body(c, inner):
            s = pl.ds(c * LANES, LANES)
            j = jref[s]
            i = iref[s]
            v = vref[s]
            g0 = plsc.load_gather(x0, [j])
            plsc.addupdate_scatter(o0, [i], v * g0)
            g1 = plsc.load_gather(x1, [j])
            plsc.addupdate_scatter(o1, [i], v * g1)
            return inner

        lax.fori_loop(0, CHUNKS, chunk_body, 0)
        pltpu.sync_copy(o0, out_hbm.at[r0])
        pltpu.sync_copy(o1, out_hbm.at[r0 + 1])
        return carry

    lax.fori_loop(0, PAIRS_PER_W, row_body, 0)


_mesh = plsc.VectorSubcoreMesh(core_axis_name="c", subcore_axis_name="s")

_call = functools.partial(
    pl.kernel,
    mesh=_mesh,
    out_type=jax.ShapeDtypeStruct((B, N), jnp.float32),
    compiler_params=pltpu.CompilerParams(needs_layout_passes=False),
    scratch_types=[
        pltpu.VMEM((NNZP,), jnp.int32),
        pltpu.VMEM((NNZP,), jnp.int32),
        pltpu.VMEM((NNZP,), jnp.float32),
        pltpu.VMEM((N,), jnp.float32),
        pltpu.VMEM((N,), jnp.float32),
        pltpu.VMEM((N,), jnp.float32),
        pltpu.VMEM((N,), jnp.float32),
        pltpu.VMEM((N,), jnp.float32),
    ],
)(_sc_kernel)


def kernel(inputs, W3, b, velocity, I, J):
    pad = NNZP - NNZ
    i_p = jnp.concatenate([I, jnp.zeros((pad,), jnp.int32)])
    j_p = jnp.concatenate([J, jnp.zeros((pad,), jnp.int32)])
    w_p = jnp.concatenate([W3, jnp.zeros((pad,), jnp.float32)])
    return _call(inputs, w_p, b, velocity, i_p, j_p)
